# baseline (speedup 1.0000x reference)
import functools

import jax
import jax.numpy as jnp
from jax import lax
from jax.experimental import pallas as pl
from jax.experimental.pallas import tpu as pltpu
from jax.experimental.pallas import tpu_sc as plsc

B, HW, M, K, C = 16, 1024, 1024, 64, 3
DECAY = 0.9
THRESHOLD = 0.05

NC, NS = 2, 16
HALF = M // 2
KH = K // 2


def _tc_body(kT_ref, mkT_ref, mu_row_ref, mu_col_ref,
             rank_s_ref, rank_mu_ref, pos_ref, valid_ref, msm_ref):
    kb = kT_ref[0]
    mkb = mkT_ref[0]
    mu_row = mu_row_ref[0]
    mu_col = mu_col_ref[0]

    logits = lax.dot_general(kb, mkb, (((0,), (0,)), ((), ())),
                             preferred_element_type=jnp.float32)
    rowmax = jnp.max(logits, axis=1, keepdims=True)
    p = jnp.exp(logits - rowmax)
    se = jnp.sum(p, axis=1, keepdims=True)
    s = p / se
    a_col = 1.0 / se
    a_row = jnp.transpose(a_col)
    msm_ref[0] = jnp.max(s, axis=0, keepdims=True)

    ii = lax.broadcasted_iota(jnp.int32, (HW, HW), 0)
    jj = lax.broadcasted_iota(jnp.int32, (HW, HW), 1)
    before = ii < jj

    take_s = (a_col < a_row) | ((a_col == a_row) & before)
    rank_s = jnp.sum(jnp.where(take_s, 1.0, 0.0), axis=0, keepdims=True)
    rank_s_ref[0] = rank_s.astype(jnp.int32)

    take_mu = (mu_col < mu_row) | ((mu_col == mu_row) & before)
    rank_mu = jnp.sum(jnp.where(take_mu, 1.0, 0.0), axis=0, keepdims=True)
    rank_mu_ref[0] = rank_mu.astype(jnp.int32)

    wv_col = a_col < THRESHOLD
    wv_row = a_row < THRESHOLD
    incl = jnp.where((ii <= jj) & wv_col, 1.0, 0.0)
    csum = jnp.sum(incl, axis=0, keepdims=True)
    countf = csum[:, HW - 1:HW]
    jrow = jj[0:1, :].astype(jnp.float32)
    posf = jnp.where(wv_row, csum - 1.0, countf + jrow - csum)
    pos_ref[0] = posf.astype(jnp.int32)
    valid_ref[0] = jnp.where(jrow < countf, 1.0, 0.0)


def _tc_stage(kT, mkT, mu_row3, mu_col3):
    out_shape = [
        jax.ShapeDtypeStruct((B, 1, HW), jnp.int32),
        jax.ShapeDtypeStruct((B, 1, M), jnp.int32),
        jax.ShapeDtypeStruct((B, 1, HW), jnp.int32),
        jax.ShapeDtypeStruct((B, 1, M), jnp.float32),
        jax.ShapeDtypeStruct((B, 1, M), jnp.float32),
    ]
    return pl.pallas_call(
        _tc_body,
        grid=(B,),
        in_specs=[
            pl.BlockSpec((1, K, HW), lambda b: (b, 0, 0)),
            pl.BlockSpec((1, K, M), lambda b: (b, 0, 0)),
            pl.BlockSpec((1, 1, M), lambda b: (b, 0, 0)),
            pl.BlockSpec((1, M, 1), lambda b: (b, 0, 0)),
        ],
        out_specs=[
            pl.BlockSpec((1, 1, HW), lambda b: (b, 0, 0)),
            pl.BlockSpec((1, 1, M), lambda b: (b, 0, 0)),
            pl.BlockSpec((1, 1, HW), lambda b: (b, 0, 0)),
            pl.BlockSpec((1, 1, M), lambda b: (b, 0, 0)),
            pl.BlockSpec((1, 1, M), lambda b: (b, 0, 0)),
        ],
        out_shape=out_shape,
    )(kT, mkT, mu_row3, mu_col3)


def _sc_body(kT_hbm, mkT_hbm, vT_hbm, mvT_hbm, mu_hbm, rkn_hbm,
             rank_s_hbm, rank_mu_hbm, pos_hbm, valid_hbm, msm_hbm,
             outkT_hbm, outvT_hbm, outu_hbm,
             rs_v, rmu_v, pos_v, val_v, msm_v, mu_v, rkn_v,
             idx2_v, idxp_v, g_v, col_v, off_v,
             slab_v, vslab_v, ko_v, ov_v, outu_v):
    cid_core = lax.axis_index("c")
    sid = lax.axis_index("s")
    wid = sid * NC + cid_core
    b = wid // 2
    h = wid % 2

    pltpu.sync_copy(rank_s_hbm.at[b], rs_v)
    pltpu.sync_copy(rank_mu_hbm.at[b], rmu_v)
    pltpu.sync_copy(pos_hbm.at[b], pos_v)
    pltpu.sync_copy(valid_hbm.at[b], val_v)
    pltpu.sync_copy(msm_hbm.at[b], msm_v)
    pltpu.sync_copy(mu_hbm.at[b], mu_v)
    pltpu.sync_copy(rkn_hbm.at[b], rkn_v)
    pltpu.sync_copy(kT_hbm.at[pl.ds(b * K + h * KH, KH)],
                    slab_v.at[pl.ds(0, KH)])
    pltpu.sync_copy(mkT_hbm.at[pl.ds(b * K + h * KH, KH)],
                    slab_v.at[pl.ds(KH, KH)])
    for c in range(C):
        pltpu.sync_copy(vT_hbm.at[c * B + b], vslab_v.at[c])
        pltpu.sync_copy(mvT_hbm.at[c * B + b], vslab_v.at[C + c])

    iota16 = lax.iota(jnp.int32, 16)
    nch = HW // 16

    for t in range(nch):
        jv = iota16 + t * 16
        plsc.store_scatter(idx2_v, [rs_v[pl.ds(t * 16, 16)]], jv)
        plsc.store_scatter(idxp_v, [rmu_v[pl.ds(t * 16, 16)]], jv)
    for t in range(nch):
        plsc.store_scatter(g_v, [pos_v[pl.ds(t * 16, 16)]],
                           idx2_v[pl.ds(t * 16, 16)])
    for t in range(nch):
        sl = pl.ds(t * 16, 16)
        vmask = val_v[sl] > 0.5
        col_v[sl] = jnp.where(vmask, g_v[sl], idxp_v[sl])
        off_v[sl] = jnp.where(vmask, 0, KH)

    for t in range(nch):
        sl = pl.ds(t * 16, 16)
        vmask = val_v[sl] > 0.5
        rk = plsc.load_gather(rkn_v, [idx2_v[sl]])
        uu = plsc.load_gather(mu_v, [idxp_v[sl]])
        outu_v[sl] = jnp.where(vmask, 1.0 + rk,
                               DECAY * uu + msm_v[sl] + rk)

    def vstep(j, carry):
        sl = pl.ds(h * HALF + j * 16, 16)
        cl16 = col_v[sl]
        vr = (off_v[sl] // KH) * C
        for c in range(C):
            vals = plsc.load_gather(vslab_v, [vr + c, cl16])
            ov_v[pl.ds(c * HALF + j * 16, 16)] = vals
        return carry
    lax.fori_loop(0, HALF // 16, vstep, 0)

    def kstep(kk, carry):
        for t in range(nch):
            sl = pl.ds(t * 16, 16)
            vals = plsc.load_gather(slab_v, [off_v[sl] + kk, col_v[sl]])
            ko_v[pl.ds(kk * M + t * 16, 16)] = vals
        return carry
    lax.fori_loop(0, KH, kstep, 0)

    pltpu.sync_copy(outu_v.at[pl.ds(h * HALF, HALF)],
                    outu_hbm.at[pl.ds(b * M + h * HALF, HALF)])
    for c in range(C):
        pltpu.sync_copy(ov_v.at[pl.ds(c * HALF, HALF)],
                        outvT_hbm.at[pl.ds((c * B + b) * M + h * HALF, HALF)])
    pltpu.sync_copy(ko_v, outkT_hbm.at[pl.ds((b * K + h * KH) * M, KH * M)])


def _sc_stage(kT2, mkT2, vT2, mvT2, m_u, rkn, rank_s, rank_mu, pos,
              validv, msm):
    mesh = plsc.VectorSubcoreMesh(core_axis_name="c", subcore_axis_name="s")
    fn = functools.partial(
        pl.kernel,
        mesh=mesh,
        compiler_params=pltpu.CompilerParams(
            needs_layout_passes=False, use_tc_tiling_on_sc=False),
        out_type=[
            jax.ShapeDtypeStruct((B * K * M,), jnp.float32),
            jax.ShapeDtypeStruct((C * B * M,), jnp.float32),
            jax.ShapeDtypeStruct((B * M,), jnp.float32),
        ],
        scratch_types=[
            pltpu.VMEM((HW,), jnp.int32),
            pltpu.VMEM((M,), jnp.int32),
            pltpu.VMEM((HW,), jnp.int32),
            pltpu.VMEM((M,), jnp.float32),
            pltpu.VMEM((M,), jnp.float32),
            pltpu.VMEM((M,), jnp.float32),
            pltpu.VMEM((HW,), jnp.float32),
            pltpu.VMEM((HW,), jnp.int32),
            pltpu.VMEM((M,), jnp.int32),
            pltpu.VMEM((HW,), jnp.int32),
            pltpu.VMEM((M,), jnp.int32),
            pltpu.VMEM((M,), jnp.int32),
            pltpu.VMEM((K, HW), jnp.float32),
            pltpu.VMEM((2 * C, HW), jnp.float32),
            pltpu.VMEM((KH * M,), jnp.float32),
            pltpu.VMEM((C * HALF,), jnp.float32),
            pltpu.VMEM((M,), jnp.float32),
        ],
    )(_sc_body)
    return fn(kT2, mkT2, vT2, mvT2, m_u, rkn, rank_s, rank_mu, pos,
              validv, msm)


def kernel(k, v, rkn_score, m_k, m_v, m_u):
    kT = jnp.transpose(k, (0, 2, 1))
    mkT = jnp.transpose(m_k, (0, 2, 1))
    vT = jnp.transpose(v, (2, 0, 1))
    mvT = jnp.transpose(m_v, (2, 0, 1))
    mu_row3 = m_u.reshape(B, 1, M)
    mu_col3 = m_u.reshape(B, M, 1)
    rank_s, rank_mu, pos, validv, msm = _tc_stage(kT, mkT, mu_row3, mu_col3)

    rkn = rkn_score[..., 0]
    outkT, outvT, outu = _sc_stage(
        kT.reshape(B * K, HW), mkT.reshape(B * K, M),
        vT.reshape(C * B, HW), mvT.reshape(C * B, M), m_u, rkn,
        rank_s.reshape(B, HW), rank_mu.reshape(B, M), pos.reshape(B, HW),
        validv.reshape(B, M), msm.reshape(B, M))
    out_mk = jnp.transpose(outkT.reshape(B, K, M), (0, 2, 1))
    out_mv = jnp.transpose(outvT.reshape(C, B, M), (1, 2, 0))
    return (out_mk, out_mv, outu.reshape(B, M))


# speedup vs baseline: 1.0023x; 1.0023x over previous
import functools

import jax
import jax.numpy as jnp
from jax import lax
from jax.experimental import pallas as pl
from jax.experimental.pallas import tpu as pltpu
from jax.experimental.pallas import tpu_sc as plsc

B, HW, M, K, C = 16, 1024, 1024, 64, 3
DECAY = 0.9
THRESHOLD = 0.05

NC, NS = 2, 16
HALF = M // 2
KH = K // 2


def _tc_body(kT_ref, mkT_ref, mu_row_ref, mu_col_ref,
             rank_s_ref, rank_mu_ref, pos_ref, valid_ref, msm_ref):
    kb = kT_ref[0]
    mkb = mkT_ref[0]
    mu_row = mu_row_ref[0]
    mu_col = mu_col_ref[0]

    logits = lax.dot_general(kb, mkb, (((0,), (0,)), ((), ())),
                             preferred_element_type=jnp.float32)
    rowmax = jnp.max(logits, axis=1, keepdims=True)
    p = jnp.exp(logits - rowmax)
    se = jnp.sum(p, axis=1, keepdims=True)
    s = p / se
    a_col = 1.0 / se
    a_row = jnp.transpose(a_col)
    msm_ref[0] = jnp.max(s, axis=0, keepdims=True)

    ii = lax.broadcasted_iota(jnp.int32, (HW, HW), 0)
    jj = lax.broadcasted_iota(jnp.int32, (HW, HW), 1)
    before = ii < jj
    ones_row = jnp.full((1, HW), 1.0, jnp.float32)

    def _colsum(mat):
        return lax.dot_general(ones_row, mat, (((1,), (0,)), ((), ())),
                               preferred_element_type=jnp.float32)

    take_s = (a_col < a_row) | ((a_col == a_row) & before)
    rank_s = _colsum(jnp.where(take_s, 1.0, 0.0))
    rank_s_ref[0] = rank_s.astype(jnp.int32)

    take_mu = (mu_col < mu_row) | ((mu_col == mu_row) & before)
    rank_mu = _colsum(jnp.where(take_mu, 1.0, 0.0))
    rank_mu_ref[0] = rank_mu.astype(jnp.int32)

    wv_col = a_col < THRESHOLD
    wv_row = a_row < THRESHOLD
    incl = jnp.where((ii <= jj) & wv_col, 1.0, 0.0)
    csum = _colsum(incl)
    countf = csum[:, HW - 1:HW]
    jrow = jj[0:1, :].astype(jnp.float32)
    posf = jnp.where(wv_row, csum - 1.0, countf + jrow - csum)
    pos_ref[0] = posf.astype(jnp.int32)
    valid_ref[0] = jnp.where(jrow < countf, 1.0, 0.0)


def _tc_stage(kT, mkT, mu_row3, mu_col3):
    out_shape = [
        jax.ShapeDtypeStruct((B, 1, HW), jnp.int32),
        jax.ShapeDtypeStruct((B, 1, M), jnp.int32),
        jax.ShapeDtypeStruct((B, 1, HW), jnp.int32),
        jax.ShapeDtypeStruct((B, 1, M), jnp.float32),
        jax.ShapeDtypeStruct((B, 1, M), jnp.float32),
    ]
    return pl.pallas_call(
        _tc_body,
        grid=(B,),
        in_specs=[
            pl.BlockSpec((1, K, HW), lambda b: (b, 0, 0)),
            pl.BlockSpec((1, K, M), lambda b: (b, 0, 0)),
            pl.BlockSpec((1, 1, M), lambda b: (b, 0, 0)),
            pl.BlockSpec((1, M, 1), lambda b: (b, 0, 0)),
        ],
        out_specs=[
            pl.BlockSpec((1, 1, HW), lambda b: (b, 0, 0)),
            pl.BlockSpec((1, 1, M), lambda b: (b, 0, 0)),
            pl.BlockSpec((1, 1, HW), lambda b: (b, 0, 0)),
            pl.BlockSpec((1, 1, M), lambda b: (b, 0, 0)),
            pl.BlockSpec((1, 1, M), lambda b: (b, 0, 0)),
        ],
        out_shape=out_shape,
    )(kT, mkT, mu_row3, mu_col3)


def _sc_body(kT_hbm, mkT_hbm, vT_hbm, mvT_hbm, mu_hbm, rkn_hbm,
             rank_s_hbm, rank_mu_hbm, pos_hbm, valid_hbm, msm_hbm,
             outkT_hbm, outvT_hbm, outu_hbm,
             rs_v, rmu_v, pos_v, val_v, msm_v, mu_v, rkn_v,
             idx2_v, idxp_v, g_v, col_v, off_v,
             slab_v, vslab_v, ko_v, ov_v, outu_v):
    cid_core = lax.axis_index("c")
    sid = lax.axis_index("s")
    wid = sid * NC + cid_core
    b = wid // 2
    h = wid % 2

    pltpu.sync_copy(rank_s_hbm.at[b], rs_v)
    pltpu.sync_copy(rank_mu_hbm.at[b], rmu_v)
    pltpu.sync_copy(pos_hbm.at[b], pos_v)
    pltpu.sync_copy(valid_hbm.at[b], val_v)
    pltpu.sync_copy(msm_hbm.at[b], msm_v)
    pltpu.sync_copy(mu_hbm.at[b], mu_v)
    pltpu.sync_copy(rkn_hbm.at[b], rkn_v)
    pltpu.sync_copy(kT_hbm.at[pl.ds(b * K + h * KH, KH)],
                    slab_v.at[pl.ds(0, KH)])
    pltpu.sync_copy(mkT_hbm.at[pl.ds(b * K + h * KH, KH)],
                    slab_v.at[pl.ds(KH, KH)])
    for c in range(C):
        pltpu.sync_copy(vT_hbm.at[c * B + b], vslab_v.at[c])
        pltpu.sync_copy(mvT_hbm.at[c * B + b], vslab_v.at[C + c])

    iota16 = lax.iota(jnp.int32, 16)
    nch = HW // 16

    for t in range(nch):
        jv = iota16 + t * 16
        plsc.store_scatter(idx2_v, [rs_v[pl.ds(t * 16, 16)]], jv)
        plsc.store_scatter(idxp_v, [rmu_v[pl.ds(t * 16, 16)]], jv)
    for t in range(nch):
        plsc.store_scatter(g_v, [pos_v[pl.ds(t * 16, 16)]],
                           idx2_v[pl.ds(t * 16, 16)])
    for t in range(nch):
        sl = pl.ds(t * 16, 16)
        vmask = val_v[sl] > 0.5
        col_v[sl] = jnp.where(vmask, g_v[sl], idxp_v[sl])
        off_v[sl] = jnp.where(vmask, 0, KH)

    for t in range(nch):
        sl = pl.ds(t * 16, 16)
        vmask = val_v[sl] > 0.5
        rk = plsc.load_gather(rkn_v, [idx2_v[sl]])
        uu = plsc.load_gather(mu_v, [idxp_v[sl]])
        outu_v[sl] = jnp.where(vmask, 1.0 + rk,
                               DECAY * uu + msm_v[sl] + rk)

    def vstep(j, carry):
        sl = pl.ds(h * HALF + j * 16, 16)
        cl16 = col_v[sl]
        vr = (off_v[sl] // KH) * C
        for c in range(C):
            vals = plsc.load_gather(vslab_v, [vr + c, cl16])
            ov_v[pl.ds(c * HALF + j * 16, 16)] = vals
        return carry
    lax.fori_loop(0, HALF // 16, vstep, 0)

    def kstep(kk, carry):
        for t in range(nch):
            sl = pl.ds(t * 16, 16)
            vals = plsc.load_gather(slab_v, [off_v[sl] + kk, col_v[sl]])
            ko_v[pl.ds(kk * M + t * 16, 16)] = vals
        return carry
    lax.fori_loop(0, KH, kstep, 0)

    pltpu.sync_copy(outu_v.at[pl.ds(h * HALF, HALF)],
                    outu_hbm.at[pl.ds(b * M + h * HALF, HALF)])
    for c in range(C):
        pltpu.sync_copy(ov_v.at[pl.ds(c * HALF, HALF)],
                        outvT_hbm.at[pl.ds((c * B + b) * M + h * HALF, HALF)])
    pltpu.sync_copy(ko_v, outkT_hbm.at[pl.ds((b * K + h * KH) * M, KH * M)])


def _sc_stage(kT2, mkT2, vT2, mvT2, m_u, rkn, rank_s, rank_mu, pos,
              validv, msm):
    mesh = plsc.VectorSubcoreMesh(core_axis_name="c", subcore_axis_name="s")
    fn = functools.partial(
        pl.kernel,
        mesh=mesh,
        compiler_params=pltpu.CompilerParams(
            needs_layout_passes=False, use_tc_tiling_on_sc=False),
        out_type=[
            jax.ShapeDtypeStruct((B * K * M,), jnp.float32),
            jax.ShapeDtypeStruct((C * B * M,), jnp.float32),
            jax.ShapeDtypeStruct((B * M,), jnp.float32),
        ],
        scratch_types=[
            pltpu.VMEM((HW,), jnp.int32),
            pltpu.VMEM((M,), jnp.int32),
            pltpu.VMEM((HW,), jnp.int32),
            pltpu.VMEM((M,), jnp.float32),
            pltpu.VMEM((M,), jnp.float32),
            pltpu.VMEM((M,), jnp.float32),
            pltpu.VMEM((HW,), jnp.float32),
            pltpu.VMEM((HW,), jnp.int32),
            pltpu.VMEM((M,), jnp.int32),
            pltpu.VMEM((HW,), jnp.int32),
            pltpu.VMEM((M,), jnp.int32),
            pltpu.VMEM((M,), jnp.int32),
            pltpu.VMEM((K, HW), jnp.float32),
            pltpu.VMEM((2 * C, HW), jnp.float32),
            pltpu.VMEM((KH * M,), jnp.float32),
            pltpu.VMEM((C * HALF,), jnp.float32),
            pltpu.VMEM((M,), jnp.float32),
        ],
    )(_sc_body)
    return fn(kT2, mkT2, vT2, mvT2, m_u, rkn, rank_s, rank_mu, pos,
              validv, msm)


def kernel(k, v, rkn_score, m_k, m_v, m_u):
    kT = jnp.transpose(k, (0, 2, 1))
    mkT = jnp.transpose(m_k, (0, 2, 1))
    vT = jnp.transpose(v, (2, 0, 1))
    mvT = jnp.transpose(m_v, (2, 0, 1))
    mu_row3 = m_u.reshape(B, 1, M)
    mu_col3 = m_u.reshape(B, M, 1)
    rank_s, rank_mu, pos, validv, msm = _tc_stage(kT, mkT, mu_row3, mu_col3)

    rkn = rkn_score[..., 0]
    outkT, outvT, outu = _sc_stage(
        kT.reshape(B * K, HW), mkT.reshape(B * K, M),
        vT.reshape(C * B, HW), mvT.reshape(C * B, M), m_u, rkn,
        rank_s.reshape(B, HW), rank_mu.reshape(B, M), pos.reshape(B, HW),
        validv.reshape(B, M), msm.reshape(B, M))
    out_mk = jnp.transpose(outkT.reshape(B, K, M), (0, 2, 1))
    out_mv = jnp.transpose(outvT.reshape(C, B, M), (1, 2, 0))
    return (out_mk, out_mv, outu.reshape(B, M))
